# baseline (speedup 1.0000x reference)
import functools

import jax
import jax.numpy as jnp
from jax import lax
from jax.experimental import pallas as pl
from jax.experimental.pallas import tpu as pltpu
from jax.experimental.pallas import tpu_sc as plsc

_B = 256
_NL = 12
_KD = 768
_NT = 10
_NP = 8
_ED = 768
_NK = 100
_ROWS = _NL * _B
_NC = 2
_NS = 16
_NW = _NC * _NS
_RPW = _ROWS // _NW
_CH = 8
_NPAIR = _RPW // (2 * _CH)


def _match_body(x_ref, keys_ref, out_ref):
    keys = keys_ref[...]
    knorm = jax.lax.dot_general(
        jnp.ones((1, _KD), jnp.float32), keys * keys,
        (((1,), (1,)), ((), ())), preferred_element_type=jnp.float32)
    colidx = jax.lax.broadcasted_iota(jnp.int32, (_B, _NK), 1)
    for l in range(_NL):
        q = x_ref[:, l, :]
        cross = jax.lax.dot_general(
            q, keys, (((1,), (1,)), ((), ())),
            preferred_element_type=jnp.float32)
        scores = knorm - 2.0 * cross
        mval = jnp.min(scores, axis=1, keepdims=True)
        idx = jnp.min(jnp.where(scores == mval, colidx, _NK),
                      axis=1, keepdims=True)
        out_ref[pl.ds(l * _B, _B), :] = idx // _NK + _NT * l


def _match_rows(x_query, task_keys):
    return pl.pallas_call(
        _match_body,
        in_specs=[
            pl.BlockSpec((_B, _NL, _KD), lambda: (0, 0, 0)),
            pl.BlockSpec((_NK, _KD), lambda: (0, 0)),
        ],
        out_specs=pl.BlockSpec((_ROWS, 1), lambda: (0, 0)),
        out_shape=jax.ShapeDtypeStruct((_ROWS, 1), jnp.int32),
    )(x_query, task_keys)


def _sc_gather(table, rows):
    mesh = plsc.VectorSubcoreMesh(core_axis_name="c", subcore_axis_name="s")

    @functools.partial(
        pl.kernel, mesh=mesh,
        out_type=jax.ShapeDtypeStruct((_NL, _B, _NP, _ED), jnp.float32),
        scratch_types=[
            pltpu.VMEM((_RPW,), jnp.int32),
            pltpu.VMEM((_CH, _NP, _ED), jnp.float32),
            pltpu.VMEM((_CH, _NP, _ED), jnp.float32),
            pltpu.SemaphoreType.DMA,
            pltpu.SemaphoreType.DMA,
        ],
    )
    def gather(table_hbm, idx_hbm, out_hbm, idx_v, hot_v, gen_v, sg, sw):
        wid = lax.axis_index("s") * _NC + lax.axis_index("c")
        base = wid * _RPW
        pltpu.sync_copy(idx_hbm.at[pl.ds(base, _RPW)], idx_v)

        def dst(c):
            r = base + c * _CH
            return out_hbm.at[r // _B, pl.ds(r % _B, _CH)]

        prev = jnp.int32(-1)
        for j in range(_NPAIR):
            vec = idx_v[pl.ds(j * 2 * _CH, 16)]
            s0 = vec[0]
            allsame = vec[1] == s0
            for i in range(2, 16):
                allsame = allsame & (vec[i] == s0)
            fresh = allsame & (s0 != prev)
            prev = jnp.where(fresh, s0, prev)

            @pl.when(fresh)
            def _():
                pltpu.async_copy(
                    table_hbm.at[idx_v.at[pl.ds(j * 2 * _CH, _CH)]],
                    hot_v, sg).wait()

            @pl.when(allsame)
            def _():
                d1 = pltpu.async_copy(hot_v, dst(2 * j), sw)
                d2 = pltpu.async_copy(hot_v, dst(2 * j + 1), sw)
                d1.wait()
                d2.wait()

            @pl.when(jnp.logical_not(allsame))
            def _():
                for c in (2 * j, 2 * j + 1):
                    pltpu.async_copy(
                        table_hbm.at[idx_v.at[pl.ds(c * _CH, _CH)]],
                        gen_v, sg).wait()
                    pltpu.async_copy(gen_v, dst(c), sw).wait()

    return gather(table, rows)


def kernel(x_query, vis_mark, P, task_keys):
    del vis_mark
    rows = _match_rows(x_query, task_keys).reshape(_ROWS)
    table = P.reshape(_NL * _NT, _NP, _ED)
    out = _sc_gather(table, rows)
    return (out, jnp.float32(0.0))


# speedup vs baseline: 1.1740x; 1.1740x over previous
import functools

import jax
import jax.numpy as jnp
from jax import lax
from jax.experimental import pallas as pl
from jax.experimental.pallas import tpu as pltpu
from jax.experimental.pallas import tpu_sc as plsc

_B = 256
_NL = 12
_KD = 768
_NT = 10
_NP = 8
_ED = 768
_NK = 100
_ROWS = _NL * _B
_NC = 2
_NS = 16
_NW = _NC * _NS
_RPW = _ROWS // _NW
_CH = 8
_NPAIR = _RPW // (2 * _CH)


def _match_body(q_ref, keys_ref, out_ref):
    q = q_ref[0]
    keys = keys_ref[...]
    knorm = jax.lax.dot_general(
        jnp.ones((1, _KD), jnp.float32), keys * keys,
        (((1,), (1,)), ((), ())), preferred_element_type=jnp.float32)
    cross = jax.lax.dot_general(
        q, keys, (((1,), (1,)), ((), ())),
        preferred_element_type=jnp.float32)
    scores = knorm - 2.0 * cross
    colidx = jax.lax.broadcasted_iota(jnp.int32, (_B, _NK), 1)
    mval = jnp.min(scores, axis=1, keepdims=True)
    idx = jnp.min(jnp.where(scores == mval, colidx, _NK), axis=1, keepdims=True)
    out_ref[...] = idx // _NK + _NT * pl.program_id(0)


def _match_rows(xq, task_keys):
    return pl.pallas_call(
        _match_body,
        grid=(_NL,),
        in_specs=[
            pl.BlockSpec((1, _B, _KD), lambda l: (l, 0, 0)),
            pl.BlockSpec((_NK, _KD), lambda l: (0, 0)),
        ],
        out_specs=pl.BlockSpec((_B, 1), lambda l: (l, 0)),
        out_shape=jax.ShapeDtypeStruct((_ROWS, 1), jnp.int32),
    )(xq, task_keys)


def _sc_gather(table, rows):
    mesh = plsc.VectorSubcoreMesh(core_axis_name="c", subcore_axis_name="s")

    @functools.partial(
        pl.kernel, mesh=mesh,
        out_type=jax.ShapeDtypeStruct((_NL, _B, _NP, _ED), jnp.float32),
        scratch_types=[
            pltpu.VMEM((_RPW,), jnp.int32),
            pltpu.VMEM((_CH, _NP, _ED), jnp.float32),
            pltpu.VMEM((_CH, _NP, _ED), jnp.float32),
            pltpu.SemaphoreType.DMA,
            pltpu.SemaphoreType.DMA,
        ],
    )
    def gather(table_hbm, idx_hbm, out_hbm, idx_v, hot_v, gen_v, sg, sw):
        wid = lax.axis_index("s") * _NC + lax.axis_index("c")
        base = wid * _RPW
        pltpu.sync_copy(idx_hbm.at[pl.ds(base, _RPW)], idx_v)

        def dst(c):
            r = base + c * _CH
            return out_hbm.at[r // _B, pl.ds(r % _B, _CH)]

        prev = jnp.int32(-1)
        for j in range(_NPAIR):
            vec = idx_v[pl.ds(j * 2 * _CH, 16)]
            s0 = vec[0]
            allsame = vec[1] == s0
            for i in range(2, 16):
                allsame = allsame & (vec[i] == s0)
            fresh = allsame & (s0 != prev)
            prev = jnp.where(fresh, s0, prev)

            @pl.when(fresh)
            def _():
                pltpu.async_copy(
                    table_hbm.at[idx_v.at[pl.ds(j * 2 * _CH, _CH)]],
                    hot_v, sg).wait()

            @pl.when(allsame)
            def _():
                d1 = pltpu.async_copy(hot_v, dst(2 * j), sw)
                d2 = pltpu.async_copy(hot_v, dst(2 * j + 1), sw)
                d1.wait()
                d2.wait()

            @pl.when(jnp.logical_not(allsame))
            def _():
                for c in (2 * j, 2 * j + 1):
                    pltpu.async_copy(
                        table_hbm.at[idx_v.at[pl.ds(c * _CH, _CH)]],
                        gen_v, sg).wait()
                    pltpu.async_copy(gen_v, dst(c), sw).wait()

    return gather(table, rows)


def kernel(x_query, vis_mark, P, task_keys):
    del vis_mark
    xq = jnp.transpose(x_query, (1, 0, 2))
    rows = _match_rows(xq, task_keys).reshape(_ROWS)
    table = P.reshape(_NL * _NT, _NP, _ED)
    out = _sc_gather(table, rows)
    return (out, jnp.float32(0.0))
